# baseline (speedup 1.0000x reference)
import functools

import jax
import jax.numpy as jnp
from jax import lax
from jax.experimental import pallas as pl
from jax.experimental.pallas import tpu as pltpu
from jax.experimental.pallas import tpu_sc as plsc

N = 8192
D_MODEL = 2048

NC, NS, L = 2, 16, 16
NW = NC * NS
TOK_PER_W = N // NW
CH = 16
NCHUNK = TOK_PER_W // CH


def _sc_combine_probe(x_hbm, out_hbm, idx0_v, idx1_v, rows0_v, rows1_v, s0, s1):
    wid = lax.axis_index("s") * NC + lax.axis_index("c")
    base = wid * TOK_PER_W

    def chunk(c, carry):
        t0 = base + c * CH
        t = lax.broadcasted_iota(jnp.int32, (L,), 0) + t0
        idx0_v[...] = (t * 2 + 1) & (N - 1)
        idx1_v[...] = (t * 7 + 3) & (N - 1)
        cp0 = pltpu.make_async_copy(x_hbm.at[idx0_v], rows0_v, s0)
        cp1 = pltpu.make_async_copy(x_hbm.at[idx1_v], rows1_v, s1)
        cp0.start()
        cp1.start()
        cp0.wait()
        cp1.wait()
        pltpu.sync_copy(rows0_v, out_hbm.at[pl.ds(t0, CH)])
        return carry

    lax.fori_loop(0, NCHUNK, chunk, 0)


@jax.jit
def kernel(x, Wg, Wd, bd, Wu, bu):
    mesh = plsc.VectorSubcoreMesh(core_axis_name="c", subcore_axis_name="s")
    f = functools.partial(
        pl.kernel,
        out_type=jax.ShapeDtypeStruct((N, D_MODEL), jnp.float32),
        mesh=mesh,
        scratch_types=[
            pltpu.VMEM((CH,), jnp.int32),
            pltpu.VMEM((CH,), jnp.int32),
            pltpu.VMEM((CH, D_MODEL), jnp.float32),
            pltpu.VMEM((CH, D_MODEL), jnp.float32),
            pltpu.SemaphoreType.DMA,
            pltpu.SemaphoreType.DMA,
        ],
    )(_sc_combine_probe)
    return f(x)


# speedup vs baseline: 1.0215x; 1.0215x over previous
import jax
import jax.numpy as jnp
from jax.experimental import pallas as pl

N = 8192
D_MODEL = 2048
BOTTLENECK = 64
NUM_EXPERTS = 8
EB = NUM_EXPERTS * BOTTLENECK

BN = 1024

_NT = (((1,), (1,)), ((), ()))


def _moe_body(x_ref, wgt_ref, wdt_ref, bdf_ref, wu_ref, bu_ref, o_ref):
    x = x_ref[...]

    logits = jax.lax.dot_general(
        x, wgt_ref[...], _NT, preferred_element_type=jnp.float32)
    mx = jnp.max(logits, axis=-1, keepdims=True)
    ex = jnp.exp(logits - mx)
    sm = ex / jnp.sum(ex, axis=-1, keepdims=True)

    col = jax.lax.broadcasted_iota(jnp.int32, (BN, NUM_EXPERTS), 1)
    m0 = jnp.max(sm, axis=-1, keepdims=True)
    e0 = jnp.min(jnp.where(sm == m0, col, NUM_EXPERTS), axis=-1, keepdims=True)
    oh0 = col == e0
    sm1 = jnp.where(oh0, -1.0, sm)
    m1 = jnp.max(sm1, axis=-1, keepdims=True)
    e1 = jnp.min(jnp.where(sm1 == m1, col, NUM_EXPERTS), axis=-1, keepdims=True)
    oh1 = col == e1
    denom = m0 + m1 + 1e-8
    m = jnp.where(oh0, m0 / denom, 0.0) + jnp.where(oh1, m1 / denom, 0.0)

    erow = jax.lax.broadcasted_iota(jnp.int32, (NUM_EXPERTS, EB), 0)
    ecol = jax.lax.broadcasted_iota(jnp.int32, (NUM_EXPERTS, EB), 1) // BOTTLENECK
    expand = (erow == ecol).astype(jnp.float32)
    m_exp = jnp.dot(m, expand, preferred_element_type=jnp.float32)

    xb = x.astype(jnp.bfloat16)
    wdb = wdt_ref[...].astype(jnp.bfloat16)
    down = jax.lax.dot_general(
        xb, wdb, _NT, preferred_element_type=jnp.float32) + bdf_ref[...]
    h = 0.5 * down * (1.0 + jax.lax.erf(down * 0.7071067811865476))
    g = (h * m_exp).astype(jnp.bfloat16)
    out = jnp.dot(g, wu_ref[...], preferred_element_type=jnp.float32)
    out = out + jnp.dot(m, bu_ref[...], preferred_element_type=jnp.float32)
    o_ref[...] = out


@jax.jit
def kernel(x, Wg, Wd, bd, Wu, bu):
    wgt = Wg
    wdt = Wd.reshape(EB, D_MODEL)
    wu = jnp.transpose(Wu, (0, 2, 1)).reshape(EB, D_MODEL).astype(jnp.bfloat16)
    bdf = bd.reshape(1, EB)

    grid = (N // BN,)
    return pl.pallas_call(
        _moe_body,
        grid=grid,
        in_specs=[
            pl.BlockSpec((BN, D_MODEL), lambda i: (i, 0)),
            pl.BlockSpec((NUM_EXPERTS, D_MODEL), lambda i: (0, 0)),
            pl.BlockSpec((EB, D_MODEL), lambda i: (0, 0)),
            pl.BlockSpec((1, EB), lambda i: (0, 0)),
            pl.BlockSpec((EB, D_MODEL), lambda i: (0, 0)),
            pl.BlockSpec((NUM_EXPERTS, D_MODEL), lambda i: (0, 0)),
        ],
        out_specs=pl.BlockSpec((BN, D_MODEL), lambda i: (i, 0)),
        out_shape=jax.ShapeDtypeStruct((N, D_MODEL), jnp.float32),
    )(x, wgt, wdt, bdf, wu, bu)
